# baseline (speedup 1.0000x reference)
import functools

import jax
import jax.numpy as jnp
from jax import lax
from jax.experimental import pallas as pl
from jax.experimental.pallas import tpu as pltpu, tpu_sc as plsc

N = 10000
E = 320000
D = 128
L = 3
R4 = 4
NC = 2
NS = 16
NW = NC * NS
CK = 128
CHW = 80
CHQ = CHW // 4
CHD = 160
E_PAD = NW * CHW * CK
NR = 10112
RPT = NR // NS
NP1 = NR
ZPT = NP1 // NS

_mesh = plsc.VectorSubcoreMesh(core_axis_name="c", subcore_axis_name="s")


@functools.partial(
    pl.kernel,
    out_type=jax.ShapeDtypeStruct((R4 * NP1,), jnp.float32),
    mesh=_mesh,
    scratch_types=[
        pltpu.VMEM((CHD, CK), jnp.int32),
        pltpu.VMEM((640,), jnp.float32),
        pltpu.VMEM((CK,), jnp.float32),
        pltpu.VMEM((640,), jnp.float32),
        pltpu.VMEM_SHARED((NP1,), jnp.float32),
        pltpu.VMEM_SHARED((NP1,), jnp.float32),
        pltpu.SemaphoreType.DMA,
    ],
)
def _deg_kernel(dst_hbm, out_hbm, didx, zbuf, ones, stage, acc_a, acc_b, dsem):
    c = lax.axis_index("c")
    s = lax.axis_index("s")
    accs = (acc_a, acc_b)
    for t in range(40):
        zbuf[pl.ds(t * 16, 16)] = jnp.zeros((16,), jnp.float32)
    for t in range(CK // 16):
        ones[pl.ds(t * 16, 16)] = jnp.ones((16,), jnp.float32)
    for k in range(2):
        pltpu.sync_copy(zbuf.at[pl.ds(0, ZPT)], accs[k].at[pl.ds(s * ZPT, ZPT)])
    plsc.subcore_barrier()
    for k in range(2):
        rel = 2 * c + k
        pltpu.sync_copy(dst_hbm.at[rel, s], didx)

        @pl.loop(0, CHD)
        def _(j):
            pltpu.async_copy(ones, accs[k].at[didx.at[j]], dsem, add=True)

        pltpu.make_async_copy(dst_hbm.at[rel, s], didx, dsem).wait()
    plsc.subcore_barrier()
    for k in range(2):
        pltpu.sync_copy(accs[k].at[pl.ds(s * ZPT, ZPT)], stage.at[pl.ds(0, ZPT)])
        pltpu.sync_copy(stage.at[pl.ds(0, ZPT)],
                        out_hbm.at[pl.ds((2 * c + k) * NP1 + s * ZPT, ZPT)])


@functools.partial(
    pl.kernel,
    out_type=jax.ShapeDtypeStruct((2, NC, NR, D), jnp.float32),
    mesh=_mesh,
    scratch_types=[
        pltpu.VMEM((CHQ, CK), jnp.int32),
        pltpu.VMEM((CHQ, CK), jnp.int32),
        pltpu.VMEM((CHQ, CK), jnp.int32),
        pltpu.VMEM((CHQ, CK), jnp.int32),
        pltpu.VMEM((CK, D), jnp.float32),
        pltpu.VMEM((CK, D), jnp.float32),
        pltpu.VMEM_SHARED((NR, D), jnp.float32),
        pltpu.SemaphoreType.DMA,
        pltpu.SemaphoreType.DMA,
        pltpu.SemaphoreType.DMA,
        pltpu.SemaphoreType.DMA,
    ],
)
def _agg2_kernel(xsa_hbm, xsb_hbm, zeros_hbm, src_hbm, dst_hbm, out_hbm,
                 sidx0, didx0, sidx1, didx1, bufA, bufB, acc,
                 gsemA, gsemB, ssem, isem):
    c = lax.axis_index("c")
    s = lax.axis_index("s")
    w = c * NS + s

    @pl.when(c == 0)
    def _():
        pltpu.sync_copy(xsa_hbm.at[pl.ds(s * RPT, RPT)],
                        acc.at[pl.ds(s * RPT, RPT)])

    @pl.when(c == 1)
    def _():
        pltpu.sync_copy(zeros_hbm.at[pl.ds(s * RPT, RPT)],
                        acc.at[pl.ds(s * RPT, RPT)])

    plsc.subcore_barrier()
    sbufs = (sidx0, sidx1)
    dbufs = (didx0, didx1)
    pltpu.sync_copy(src_hbm.at[0, w, 0], sidx0)
    pltpu.sync_copy(dst_hbm.at[0, w, 0], didx0)
    for seg in range(8):
        k, q = divmod(seg, 4)
        si = sbufs[seg % 2]
        di = dbufs[seg % 2]
        table = xsa_hbm if k == 0 else xsb_hbm
        if seg < 7:
            nk, nq = divmod(seg + 1, 4)
            pltpu.async_copy(src_hbm.at[nk, w, nq], sbufs[(seg + 1) % 2], isem)
            pltpu.async_copy(dst_hbm.at[nk, w, nq], dbufs[(seg + 1) % 2], isem)
        pltpu.async_copy(table.at[si.at[0]], bufA, gsemA)

        @pl.loop(0, CHQ // 2)
        def _(j):
            a = 2 * j
            b = 2 * j + 1
            pltpu.async_copy(table.at[si.at[b]], bufB, gsemB)
            pltpu.make_async_copy(table.at[si.at[a]], bufA, gsemA).wait()
            pltpu.async_copy(bufA, acc.at[di.at[a]], ssem, add=True).wait()

            @pl.when(j < CHQ // 2 - 1)
            def _():
                pltpu.async_copy(table.at[si.at[a + 2]], bufA, gsemA)

            pltpu.make_async_copy(table.at[si.at[b]], bufB, gsemB).wait()
            pltpu.async_copy(bufB, acc.at[di.at[b]], ssem, add=True).wait()

        if seg in (3, 7):
            plsc.subcore_barrier()
            pltpu.sync_copy(acc.at[pl.ds(s * RPT, RPT)],
                            out_hbm.at[k, c, pl.ds(s * RPT, RPT)])
            plsc.subcore_barrier()
        if seg < 7:
            nk, nq = divmod(seg + 1, 4)
            pltpu.make_async_copy(src_hbm.at[nk, w, nq],
                                  sbufs[(seg + 1) % 2], isem).wait()
            pltpu.make_async_copy(dst_hbm.at[nk, w, nq],
                                  dbufs[(seg + 1) % 2], isem).wait()


def _krv_body(re_ref, wr_ref, br_ref, out_ref):
    r = re_ref[...]
    out_ref[0] = r
    br = br_ref[...]
    for i in range(L):
        r = (jnp.dot(r, wr_ref[i], preferred_element_type=jnp.float32)
             + br[i:i + 1, :])
        out_ref[i + 1] = r


def _rv_chain(rel_emb, W_rel, b_rel):
    return pl.pallas_call(
        _krv_body,
        out_shape=jax.ShapeDtypeStruct((L + 1, R4, D), jnp.float32),
    )(rel_emb, W_rel, b_rel)


def _kpre_body(emb_ref, rv_ref, w_ref, dv_ref, xs_ref):
    e = emb_ref[...]
    row = rv_ref[...].reshape(1, D)
    xw = jnp.dot(e * row, w_ref[...], preferred_element_type=jnp.float32)
    y = xw * dv_ref[...]
    pad = jnp.zeros((NR - N, D), jnp.float32)
    xs_ref[...] = jnp.concatenate([y, pad], axis=0)


def _kpre(emb, rv_r, W_i, dv_r):
    return pl.pallas_call(
        _kpre_body,
        out_shape=jax.ShapeDtypeStruct((NR, D), jnp.float32),
    )(emb, rv_r, W_i, dv_r)


def _bn_tail(a, emb_ref, dv_ref, b_ref, g_ref, bt_ref, out_ref):
    z = a * dv_ref[...] + b_ref[...]
    m = jnp.mean(z, axis=0, keepdims=True)
    v = jnp.mean(z * z, axis=0, keepdims=True) - m * m
    h = (z - m) * lax.rsqrt(v + 1e-5) * g_ref[...] + bt_ref[...]
    h = jnp.where(h >= 0, h, 0.01 * h)
    out_ref[...] = emb_ref[...] + h


def _kbn_a_body(agg_ref, emb_ref, dv_ref, b_ref, g_ref, bt_ref, out_ref):
    ag = agg_ref[...]
    a = ag[0, 0, :N] + ag[0, 1, :N]
    _bn_tail(a, emb_ref, dv_ref, b_ref, g_ref, bt_ref, out_ref)


def _kbn_b_body(agg_ref, xsb_ref, emb_ref, dv_ref, b_ref, g_ref, bt_ref,
                out_ref):
    ag = agg_ref[...]
    a = (ag[1, 0, :N] + ag[1, 1, :N] - ag[0, 0, :N] - ag[0, 1, :N]
         + xsb_ref[...][:N])
    _bn_tail(a, emb_ref, dv_ref, b_ref, g_ref, bt_ref, out_ref)


def _kbn_a(agg, emb, dv_r, b_i, g_i, bt_i):
    return pl.pallas_call(
        _kbn_a_body,
        out_shape=jax.ShapeDtypeStruct((N, D), jnp.float32),
    )(agg, emb, dv_r, b_i, g_i, bt_i)


def _kbn_b(agg, xsb, emb, dv_r, b_i, g_i, bt_i):
    return pl.pallas_call(
        _kbn_b_body,
        out_shape=jax.ShapeDtypeStruct((N, D), jnp.float32),
    )(agg, xsb, emb, dv_r, b_i, g_i, bt_i)


def _kpost_a_body(agg_ref, dv_ref, b_ref, out_ref):
    ag = agg_ref[...]
    a = ag[0, 0, :N] + ag[0, 1, :N]
    out_ref[...] = a * dv_ref[...] + b_ref[...]


def _kpost_b_body(agg_ref, xsb_ref, dv_ref, b_ref, out_ref):
    ag = agg_ref[...]
    a = (ag[1, 0, :N] + ag[1, 1, :N] - ag[0, 0, :N] - ag[0, 1, :N]
         + xsb_ref[...][:N])
    out_ref[...] = a * dv_ref[...] + b_ref[...]


def _kpost_a(agg, dv_r, b_i):
    return pl.pallas_call(
        _kpost_a_body,
        out_shape=jax.ShapeDtypeStruct((N, D), jnp.float32),
    )(agg, dv_r, b_i)


def _kpost_b(agg, xsb, dv_r, b_i):
    return pl.pallas_call(
        _kpost_b_body,
        out_shape=jax.ShapeDtypeStruct((N, D), jnp.float32),
    )(agg, xsb, dv_r, b_i)


def kernel(features, rel_emb, edge_index, W_gcn, b_gcn, bn_gamma, bn_beta,
           W_rel, b_rel, is_training):
    del is_training

    pad = E_PAD - E
    ar = jnp.arange(pad, dtype=jnp.int32)
    pad_src = (ar * 911) % N
    pad_dst = N + (ar % 64)
    src_flat = jnp.concatenate(
        [edge_index[:, 0, :], jnp.broadcast_to(pad_src, (R4, pad))], axis=1)
    dst_flat = jnp.concatenate(
        [edge_index[:, 1, :], jnp.broadcast_to(pad_dst, (R4, pad))], axis=1)
    src_w = src_flat.reshape(R4, NW, CHW, CK)
    dst_w = dst_flat.reshape(R4, NW, CHW, CK)
    dst_t = dst_flat.reshape(R4, NS, CHD, CK)
    zeros_nr = jnp.zeros((NR, D), jnp.float32)

    rvs = _rv_chain(rel_emb, W_rel, b_rel)

    parts = _deg_kernel(dst_t).reshape(R4, NP1)
    dinv3 = lax.rsqrt(parts[:, :N] + 1.0)[:, :, None]
    dvs = [dinv3[r] for r in range(R4)]

    b2d = [b_gcn[i][None, :] for i in range(L)]
    g2d = [bn_gamma[i][None, :] for i in range(L - 1)]
    bt2d = [bn_beta[i][None, :] for i in range(L - 1)]
    src_g = src_w.reshape(2, 2, NW, 4, CHQ, CK)
    dst_g = dst_w.reshape(2, 2, NW, 4, CHQ, CK)

    embs = [features] * R4
    for i in range(L - 1):
        xss = [_kpre(embs[r], rvs[i, r:r + 1], W_gcn[i], dvs[r])
               for r in range(R4)]
        aggs = [_agg2_kernel(xss[2 * g], xss[2 * g + 1], zeros_nr,
                             src_g[g], dst_g[g]) for g in range(2)]
        embs = [
            _kbn_a(aggs[0], embs[0], dvs[0], b2d[i], g2d[i], bt2d[i]),
            _kbn_b(aggs[0], xss[1], embs[1], dvs[1], b2d[i], g2d[i], bt2d[i]),
            _kbn_a(aggs[1], embs[2], dvs[2], b2d[i], g2d[i], bt2d[i]),
            _kbn_b(aggs[1], xss[3], embs[3], dvs[3], b2d[i], g2d[i], bt2d[i]),
        ]
    xss = [_kpre(embs[r], rvs[L - 1, r:r + 1], W_gcn[L - 1], dvs[r])
           for r in range(R4)]
    aggs = [_agg2_kernel(xss[2 * g], xss[2 * g + 1], zeros_nr,
                         src_g[g], dst_g[g]) for g in range(2)]
    embf = [
        _kpost_a(aggs[0], dvs[0], b2d[L - 1]),
        _kpost_b(aggs[0], xss[1], dvs[1], b2d[L - 1]),
        _kpost_a(aggs[1], dvs[2], b2d[L - 1]),
        _kpost_b(aggs[1], xss[3], dvs[3], b2d[L - 1]),
    ]

    rf = rvs[L]
    return (embf[3], embf[0], embf[1], embf[2],
            rf[3], rf[0], rf[1], rf[2])


# speedup vs baseline: 1.0163x; 1.0163x over previous
import functools

import jax
import jax.numpy as jnp
from jax import lax
from jax.experimental import pallas as pl
from jax.experimental.pallas import tpu as pltpu, tpu_sc as plsc

N = 10000
E = 320000
D = 128
L = 3
R4 = 4
NC = 2
NS = 16
NW = NC * NS
CK = 128
CHW = 80
CHH = CHW // 2
CHD = 160
E_PAD = NW * CHW * CK
NR = 10112
RPT = NR // NS
NP1 = NR
ZPT = NP1 // NS

_mesh = plsc.VectorSubcoreMesh(core_axis_name="c", subcore_axis_name="s")


@functools.partial(
    pl.kernel,
    out_type=jax.ShapeDtypeStruct((R4 * NP1,), jnp.float32),
    mesh=_mesh,
    scratch_types=[
        pltpu.VMEM((CHD, CK), jnp.int32),
        pltpu.VMEM((640,), jnp.float32),
        pltpu.VMEM((CK,), jnp.float32),
        pltpu.VMEM((640,), jnp.float32),
        pltpu.VMEM_SHARED((NP1,), jnp.float32),
        pltpu.VMEM_SHARED((NP1,), jnp.float32),
        pltpu.SemaphoreType.DMA,
    ],
)
def _deg_kernel(dst_hbm, out_hbm, didx, zbuf, ones, stage, acc_a, acc_b, dsem):
    c = lax.axis_index("c")
    s = lax.axis_index("s")
    accs = (acc_a, acc_b)
    for t in range(40):
        zbuf[pl.ds(t * 16, 16)] = jnp.zeros((16,), jnp.float32)
    for t in range(CK // 16):
        ones[pl.ds(t * 16, 16)] = jnp.ones((16,), jnp.float32)
    for k in range(2):
        pltpu.sync_copy(zbuf.at[pl.ds(0, ZPT)], accs[k].at[pl.ds(s * ZPT, ZPT)])
    plsc.subcore_barrier()
    for k in range(2):
        rel = 2 * c + k
        pltpu.sync_copy(dst_hbm.at[rel, s], didx)

        @pl.loop(0, CHD)
        def _(j):
            pltpu.async_copy(ones, accs[k].at[didx.at[j]], dsem, add=True)

        pltpu.make_async_copy(dst_hbm.at[rel, s], didx, dsem).wait()
    plsc.subcore_barrier()
    for k in range(2):
        pltpu.sync_copy(accs[k].at[pl.ds(s * ZPT, ZPT)], stage.at[pl.ds(0, ZPT)])
        pltpu.sync_copy(stage.at[pl.ds(0, ZPT)],
                        out_hbm.at[pl.ds((2 * c + k) * NP1 + s * ZPT, ZPT)])


@functools.partial(
    pl.kernel,
    out_type=jax.ShapeDtypeStruct((2, NC, NR, D), jnp.float32),
    mesh=_mesh,
    scratch_types=[
        pltpu.VMEM((CHH, CK), jnp.int32),
        pltpu.VMEM((CHH, CK), jnp.int32),
        pltpu.VMEM((CK, D), jnp.float32),
        pltpu.VMEM((CK, D), jnp.float32),
        pltpu.VMEM_SHARED((NR, D), jnp.float32),
        pltpu.SemaphoreType.DMA,
        pltpu.SemaphoreType.DMA,
        pltpu.SemaphoreType.DMA,
    ],
)
def _agg2_kernel(xsa_hbm, xsb_hbm, zeros_hbm, src_hbm, dst_hbm, out_hbm,
                 sidx, didx, bufA, bufB, acc, gsemA, gsemB, ssem):
    c = lax.axis_index("c")
    s = lax.axis_index("s")
    w = c * NS + s

    @pl.when(c == 0)
    def _():
        pltpu.sync_copy(xsa_hbm.at[pl.ds(s * RPT, RPT)],
                        acc.at[pl.ds(s * RPT, RPT)])

    @pl.when(c == 1)
    def _():
        pltpu.sync_copy(zeros_hbm.at[pl.ds(s * RPT, RPT)],
                        acc.at[pl.ds(s * RPT, RPT)])

    plsc.subcore_barrier()
    for k in range(2):
        table = xsa_hbm if k == 0 else xsb_hbm
        for half in range(2):
            pltpu.sync_copy(src_hbm.at[k, w, pl.ds(half * CHH, CHH)], sidx)
            pltpu.sync_copy(dst_hbm.at[k, w, pl.ds(half * CHH, CHH)], didx)
            pltpu.async_copy(table.at[sidx.at[0]], bufA, gsemA)

            @pl.loop(0, CHH // 2)
            def _(j):
                a = 2 * j
                b = 2 * j + 1
                pltpu.async_copy(table.at[sidx.at[b]], bufB, gsemB)
                pltpu.make_async_copy(table.at[sidx.at[a]], bufA, gsemA).wait()
                pltpu.async_copy(bufA, acc.at[didx.at[a]], ssem, add=True).wait()

                @pl.when(j < CHH // 2 - 1)
                def _():
                    pltpu.async_copy(table.at[sidx.at[a + 2]], bufA, gsemA)

                pltpu.make_async_copy(table.at[sidx.at[b]], bufB, gsemB).wait()
                pltpu.async_copy(bufB, acc.at[didx.at[b]], ssem, add=True).wait()

        plsc.subcore_barrier()
        pltpu.sync_copy(acc.at[pl.ds(s * RPT, RPT)],
                        out_hbm.at[k, c, pl.ds(s * RPT, RPT)])
        plsc.subcore_barrier()


def _krv_body(re_ref, wr_ref, br_ref, out_ref):
    r = re_ref[...]
    out_ref[0] = r
    br = br_ref[...]
    for i in range(L):
        r = (jnp.dot(r, wr_ref[i], preferred_element_type=jnp.float32)
             + br[i:i + 1, :])
        out_ref[i + 1] = r


def _rv_chain(rel_emb, W_rel, b_rel):
    return pl.pallas_call(
        _krv_body,
        out_shape=jax.ShapeDtypeStruct((L + 1, R4, D), jnp.float32),
    )(rel_emb, W_rel, b_rel)


def _kpre_body(emb_ref, rv_ref, w_ref, dv_ref, xs_ref):
    e = emb_ref[...]
    row = rv_ref[...].reshape(1, D)
    xw = jnp.dot(e * row, w_ref[...], preferred_element_type=jnp.float32)
    y = xw * dv_ref[...]
    pad = jnp.zeros((NR - N, D), jnp.float32)
    xs_ref[...] = jnp.concatenate([y, pad], axis=0)


def _kpre(emb, rv_r, W_i, dv_r):
    return pl.pallas_call(
        _kpre_body,
        out_shape=jax.ShapeDtypeStruct((NR, D), jnp.float32),
    )(emb, rv_r, W_i, dv_r)


def _bn_tail(a, emb_ref, dv_ref, b_ref, g_ref, bt_ref, out_ref):
    z = a * dv_ref[...] + b_ref[...]
    m = jnp.mean(z, axis=0, keepdims=True)
    v = jnp.mean(z * z, axis=0, keepdims=True) - m * m
    h = (z - m) * lax.rsqrt(v + 1e-5) * g_ref[...] + bt_ref[...]
    h = jnp.where(h >= 0, h, 0.01 * h)
    out_ref[...] = emb_ref[...] + h


def _kbn_a_body(agg_ref, emb_ref, dv_ref, b_ref, g_ref, bt_ref, out_ref):
    ag = agg_ref[...]
    a = ag[0, 0, :N] + ag[0, 1, :N]
    _bn_tail(a, emb_ref, dv_ref, b_ref, g_ref, bt_ref, out_ref)


def _kbn_b_body(agg_ref, xsb_ref, emb_ref, dv_ref, b_ref, g_ref, bt_ref,
                out_ref):
    ag = agg_ref[...]
    a = (ag[1, 0, :N] + ag[1, 1, :N] - ag[0, 0, :N] - ag[0, 1, :N]
         + xsb_ref[...][:N])
    _bn_tail(a, emb_ref, dv_ref, b_ref, g_ref, bt_ref, out_ref)


def _kbn_a(agg, emb, dv_r, b_i, g_i, bt_i):
    return pl.pallas_call(
        _kbn_a_body,
        out_shape=jax.ShapeDtypeStruct((N, D), jnp.float32),
    )(agg, emb, dv_r, b_i, g_i, bt_i)


def _kbn_b(agg, xsb, emb, dv_r, b_i, g_i, bt_i):
    return pl.pallas_call(
        _kbn_b_body,
        out_shape=jax.ShapeDtypeStruct((N, D), jnp.float32),
    )(agg, xsb, emb, dv_r, b_i, g_i, bt_i)


def _kpost_a_body(agg_ref, dv_ref, b_ref, out_ref):
    ag = agg_ref[...]
    a = ag[0, 0, :N] + ag[0, 1, :N]
    out_ref[...] = a * dv_ref[...] + b_ref[...]


def _kpost_b_body(agg_ref, xsb_ref, dv_ref, b_ref, out_ref):
    ag = agg_ref[...]
    a = (ag[1, 0, :N] + ag[1, 1, :N] - ag[0, 0, :N] - ag[0, 1, :N]
         + xsb_ref[...][:N])
    out_ref[...] = a * dv_ref[...] + b_ref[...]


def _kpost_a(agg, dv_r, b_i):
    return pl.pallas_call(
        _kpost_a_body,
        out_shape=jax.ShapeDtypeStruct((N, D), jnp.float32),
    )(agg, dv_r, b_i)


def _kpost_b(agg, xsb, dv_r, b_i):
    return pl.pallas_call(
        _kpost_b_body,
        out_shape=jax.ShapeDtypeStruct((N, D), jnp.float32),
    )(agg, xsb, dv_r, b_i)


def kernel(features, rel_emb, edge_index, W_gcn, b_gcn, bn_gamma, bn_beta,
           W_rel, b_rel, is_training):
    del is_training

    pad = E_PAD - E
    ar = jnp.arange(pad, dtype=jnp.int32)
    pad_src = (ar * 911) % N
    pad_dst = N + (ar % 64)
    src_flat = jnp.concatenate(
        [edge_index[:, 0, :], jnp.broadcast_to(pad_src, (R4, pad))], axis=1)
    dst_flat = jnp.concatenate(
        [edge_index[:, 1, :], jnp.broadcast_to(pad_dst, (R4, pad))], axis=1)
    src_w = src_flat.reshape(R4, NW, CHW, CK)
    dst_w = dst_flat.reshape(R4, NW, CHW, CK)
    dst_t = dst_flat.reshape(R4, NS, CHD, CK)
    zeros_nr = jnp.zeros((NR, D), jnp.float32)

    rvs = _rv_chain(rel_emb, W_rel, b_rel)

    parts = _deg_kernel(dst_t).reshape(R4, NP1)
    dinv3 = lax.rsqrt(parts[:, :N] + 1.0)[:, :, None]
    dvs = [dinv3[r] for r in range(R4)]

    b2d = [b_gcn[i][None, :] for i in range(L)]
    g2d = [bn_gamma[i][None, :] for i in range(L - 1)]
    bt2d = [bn_beta[i][None, :] for i in range(L - 1)]
    src_g = src_w.reshape(2, 2, NW, CHW, CK)
    dst_g = dst_w.reshape(2, 2, NW, CHW, CK)

    embs = [features] * R4
    for i in range(L - 1):
        xss = [_kpre(embs[r], rvs[i, r:r + 1], W_gcn[i], dvs[r])
               for r in range(R4)]
        aggs = [_agg2_kernel(xss[2 * g], xss[2 * g + 1], zeros_nr,
                             src_g[g], dst_g[g]) for g in range(2)]
        embs = [
            _kbn_a(aggs[0], embs[0], dvs[0], b2d[i], g2d[i], bt2d[i]),
            _kbn_b(aggs[0], xss[1], embs[1], dvs[1], b2d[i], g2d[i], bt2d[i]),
            _kbn_a(aggs[1], embs[2], dvs[2], b2d[i], g2d[i], bt2d[i]),
            _kbn_b(aggs[1], xss[3], embs[3], dvs[3], b2d[i], g2d[i], bt2d[i]),
        ]
    xss = [_kpre(embs[r], rvs[L - 1, r:r + 1], W_gcn[L - 1], dvs[r])
           for r in range(R4)]
    aggs = [_agg2_kernel(xss[2 * g], xss[2 * g + 1], zeros_nr,
                         src_g[g], dst_g[g]) for g in range(2)]
    embf = [
        _kpost_a(aggs[0], dvs[0], b2d[L - 1]),
        _kpost_b(aggs[0], xss[1], dvs[1], b2d[L - 1]),
        _kpost_a(aggs[1], dvs[2], b2d[L - 1]),
        _kpost_b(aggs[1], xss[3], dvs[3], b2d[L - 1]),
    ]

    rf = rvs[L]
    return (embf[3], embf[0], embf[1], embf[2],
            rf[3], rf[0], rf[1], rf[2])
